# baseline (speedup 1.0000x reference)
import jax
import jax.numpy as jnp
import numpy as np
from jax.experimental import pallas as pl
from jax.experimental.pallas import tpu as pltpu

POOL_N = 128
BATCH_N = 32
ROW = 3 * 256 * 256
SUB = ROW // 128
LANE = 128

_PROB = [True, False, True, True, True, True, True, False, False, True, True,
         True, True, True, False, False, True, True, False, True, False, True,
         False, True, True, True, True, True, True, False, True, False]
_INDEX = [83, 2, 65, 73, 78, 32, 15, 10, 71, 48, 85, 25, 116, 109, 114, 115,
          77, 28, 106, 93, 92, 0, 82, 49, 69, 87, 89, 104, 75, 4, 90, 60]


def _build_tables():
    row_to_b = {idx: b for b, idx in enumerate(_INDEX) if _PROB[b]}
    first_b = next(b for b in range(BATCH_N) if _PROB[b])
    first_keep = next(r for r in range(POOL_N) if r not in row_to_b)
    flags_p, isrc_p, psrc_p = [], [], []
    cur_b, cur_r = first_b, first_keep
    for r in range(POOL_N):
        if r in row_to_b:
            cur_b = row_to_b[r]
            flags_p.append(1)
        else:
            cur_r = r
            flags_p.append(0)
        isrc_p.append(cur_b)
        psrc_p.append(cur_r)

    first_swap_r = _INDEX[first_b]
    first_keep_b = next(b for b in range(BATCH_N) if not _PROB[b])
    flags_i, isrc_i, psrc_i = [], [], []
    cur_b, cur_r = first_keep_b, first_swap_r
    for b in range(BATCH_N):
        if _PROB[b]:
            cur_r = _INDEX[b]
            flags_i.append(0)
        else:
            cur_b = b
            flags_i.append(1)
        isrc_i.append(cur_b)
        psrc_i.append(cur_r)
    return (np.array(flags_p, np.int32), np.array(isrc_p, np.int32),
            np.array(psrc_p, np.int32), np.array(flags_i, np.int32),
            np.array(isrc_i, np.int32), np.array(psrc_i, np.int32))


(_FLAGS_P, _ISRC_P, _PSRC_P, _FLAGS_I, _ISRC_I, _PSRC_I) = _build_tables()


def _row_select_body(flags_ref, psrc_ref, isrc_ref, pool_ref, img_ref, out_ref):
    step = pl.program_id(0)
    flag = flags_ref[step]

    @pl.when(flag != 0)
    def _():
        out_ref[...] = img_ref[...]

    @pl.when(flag == 0)
    def _():
        out_ref[...] = pool_ref[...]


def _row_select_call(n_out, flags, psrc, isrc, pool3, img3):
    grid_spec = pltpu.PrefetchScalarGridSpec(
        num_scalar_prefetch=3,
        grid=(n_out,),
        in_specs=[
            pl.BlockSpec((1, SUB, LANE), lambda s, f, p, i: (p[s], 0, 0)),
            pl.BlockSpec((1, SUB, LANE), lambda s, f, p, i: (i[s], 0, 0)),
        ],
        out_specs=pl.BlockSpec((1, SUB, LANE), lambda s, f, p, i: (s, 0, 0)),
    )
    return pl.pallas_call(
        _row_select_body,
        grid_spec=grid_spec,
        out_shape=jax.ShapeDtypeStruct((n_out, SUB, LANE), jnp.float32),
    )(flags, psrc, isrc, pool3, img3)


def kernel(images, pool):
    img3 = images.reshape(BATCH_N, SUB, LANE)
    pool3 = pool.reshape(POOL_N, SUB, LANE)
    new_pool = _row_select_call(
        POOL_N, jnp.asarray(_FLAGS_P), jnp.asarray(_PSRC_P),
        jnp.asarray(_ISRC_P), pool3, img3)
    out_images = _row_select_call(
        BATCH_N, jnp.asarray(_FLAGS_I), jnp.asarray(_PSRC_I),
        jnp.asarray(_ISRC_I), pool3, img3)
    return (out_images.reshape(BATCH_N, 3, 256, 256),
            new_pool.reshape(POOL_N, 3, 256, 256))
